# baseline (speedup 1.0000x reference)
import jax
import jax.numpy as jnp
from jax.experimental import pallas as pl
from jax.experimental.pallas import tpu as pltpu

_F_SCORE = 1.2447
_BETA = 0.999


def _homovar_kernel(logits_ref, labels_ref, counts_ref, features_hbm,
                    out_ref, f_vmem, sem):
    cp = pltpu.make_async_copy(features_hbm, f_vmem, sem)
    cp.start()

    lab_row = labels_ref[:].reshape(1, -1)
    counts_row = counts_ref[:].reshape(1, -1)
    b_sz = lab_row.shape[1]
    k = counts_row.shape[1]

    lab_col = lab_row.T
    oh = (lab_col == jax.lax.broadcasted_iota(jnp.int32, (b_sz, k), 1)
          ).astype(jnp.float32)
    oht = (lab_row == jax.lax.broadcasted_iota(jnp.int32, (k, b_sz), 0)
           ).astype(jnp.float32)
    inv_counts = 1.0 / counts_row
    ones_k = jnp.ones((k, 1), jnp.float32)

    lg = logits_ref[:]
    mx = jnp.max(lg, axis=1, keepdims=True)
    e = jnp.exp(lg - mx)
    se = jnp.sum(e, axis=1, keepdims=True)
    pred = e / se
    log_p = jnp.maximum(lg - mx - jnp.log(se), -100.0)
    log_1mp = jnp.maximum(jnp.log(1.0 - pred), -100.0)
    bce = oh * (log_1mp - log_p) - log_1mp
    bcesum = jnp.dot(bce, ones_k,
                     preferred_element_type=jnp.float32)

    cp.wait()
    f = f_vmem[:]
    ohm = oht * inv_counts.T
    cmean = jnp.dot(ohm, f, preferred_element_type=jnp.float32)
    m = jnp.dot(oh, cmean, preferred_element_type=jnp.float32)
    t = jnp.abs(f - m) * (f != 0.0).astype(jnp.float32)
    ones_d = jnp.ones((f.shape[1], 1), jnp.float32)
    z = jnp.dot(t, ones_d, preferred_element_type=jnp.float32)

    zmask = (z != 0.0).astype(jnp.float32)
    zz = jnp.concatenate([z, zmask], axis=1)
    y = jax.lax.dot_general(zz, oht, (((0,), (1,)), ((), ())),
                            preferred_element_type=jnp.float32)
    s = y[0:1, :]
    nz = y[1:2, :]

    zi_mean = s * inv_counts
    z_mean = jnp.sum(zi_mean) / k
    n_total = jnp.sum(counts_row)

    ssw = (jnp.sum(z * z) - 2.0 * jnp.sum(zi_mean * s)
           + jnp.sum(zi_mean * zi_mean * nz)) / (n_total - k)
    sb = (zi_mean - z_mean) ** 2 * counts_row
    ssb = jnp.sum(sb) / (k - 1)

    cq = _F_SCORE * ssw * (k - 1) - (ssb * (k - 1) - sb)
    a = z_mean ** 2
    b = -(2.0 * z_mean * s + cq)
    cc = s ** 2
    disc = jnp.sqrt(b * b - 4.0 * a * cc)
    n_lb = jnp.abs((-b - disc) / (2.0 * a))
    n_ub = jnp.abs((-b + disc) / (2.0 * a))

    beta = jnp.where(
        counts_row < n_lb,
        jnp.power(_BETA, 1.0 / (n_lb - counts_row)),
        jnp.where(counts_row > n_ub,
                  jnp.power(_BETA, 1.0 / (counts_row - n_ub)),
                  _BETA))
    eff = 1.0 - jnp.power(beta, counts_row)
    w_cls = (1.0 - beta) / eff
    w_cls = w_cls / jnp.sum(w_cls) * k
    w_n = jnp.dot(w_cls, oht,
                  preferred_element_type=jnp.float32)

    total = jax.lax.dot_general(w_n, bcesum, (((1,), (0,)), ((), ())),
                                preferred_element_type=jnp.float32)
    out_ref[...] = total[0, 0] / (b_sz * k)


def kernel(logits, labels, features, sample_num_per_cls):
    bsz, d = features.shape
    out = pl.pallas_call(
        _homovar_kernel,
        in_specs=[
            pl.BlockSpec(memory_space=pltpu.VMEM),
            pl.BlockSpec(memory_space=pltpu.VMEM),
            pl.BlockSpec(memory_space=pltpu.VMEM),
            pl.BlockSpec(memory_space=pltpu.HBM),
        ],
        scratch_shapes=[
            pltpu.VMEM((bsz, d), jnp.float32),
            pltpu.SemaphoreType.DMA,
        ],
        out_shape=jax.ShapeDtypeStruct((), jnp.float32),
        out_specs=pl.BlockSpec(memory_space=pltpu.SMEM),
    )(logits, labels.astype(jnp.int32), sample_num_per_cls, features)
    return out


# speedup vs baseline: 1.0836x; 1.0836x over previous
import jax
import jax.numpy as jnp
from jax.experimental import pallas as pl
from jax.experimental.pallas import tpu as pltpu

_F_SCORE = 1.2447
_BETA = 0.999


def _homovar_kernel(logits_ref, labels_ref, counts_ref, features_ref,
                    out_ref):

    lab_row = labels_ref[:].reshape(1, -1)
    counts_row = counts_ref[:].reshape(1, -1)
    b_sz = lab_row.shape[1]
    k = counts_row.shape[1]

    lab_col = lab_row.T
    oh = (lab_col == jax.lax.broadcasted_iota(jnp.int32, (b_sz, k), 1)
          ).astype(jnp.float32)
    oht = (lab_row == jax.lax.broadcasted_iota(jnp.int32, (k, b_sz), 0)
           ).astype(jnp.float32)
    inv_counts = 1.0 / counts_row
    ones_k = jnp.ones((k, 1), jnp.float32)

    lg = logits_ref[:]
    mx = jnp.max(lg, axis=1, keepdims=True)
    e = jnp.exp(lg - mx)
    se = jnp.sum(e, axis=1, keepdims=True)
    pred = e / se
    log_p = jnp.maximum(lg - mx - jnp.log(se), -100.0)
    log_1mp = jnp.maximum(jnp.log(1.0 - pred), -100.0)
    bce = oh * (log_1mp - log_p) - log_1mp
    bcesum = jnp.dot(bce, ones_k,
                     preferred_element_type=jnp.float32)

    f = features_ref[:]
    ohm = oht * inv_counts.T
    cmean = jnp.dot(ohm, f, preferred_element_type=jnp.float32)
    m = jnp.dot(oh, cmean, preferred_element_type=jnp.float32)
    t = jnp.abs(f - m) * (f != 0.0).astype(jnp.float32)
    ones_d = jnp.ones((f.shape[1], 1), jnp.float32)
    z = jnp.dot(t, ones_d, preferred_element_type=jnp.float32)

    zmask = (z != 0.0).astype(jnp.float32)
    zz = jnp.concatenate([z, zmask], axis=1)
    y = jax.lax.dot_general(zz, oht, (((0,), (1,)), ((), ())),
                            preferred_element_type=jnp.float32)
    s = y[0:1, :]
    nz = y[1:2, :]

    zi_mean = s * inv_counts
    z_mean = jnp.sum(zi_mean) / k
    n_total = jnp.sum(counts_row)

    ssw = (jnp.sum(z * z) - 2.0 * jnp.sum(zi_mean * s)
           + jnp.sum(zi_mean * zi_mean * nz)) / (n_total - k)
    sb = (zi_mean - z_mean) ** 2 * counts_row
    ssb = jnp.sum(sb) / (k - 1)

    cq = _F_SCORE * ssw * (k - 1) - (ssb * (k - 1) - sb)
    a = z_mean ** 2
    b = -(2.0 * z_mean * s + cq)
    cc = s ** 2
    disc = jnp.sqrt(b * b - 4.0 * a * cc)
    n_lb = jnp.abs((-b - disc) / (2.0 * a))
    n_ub = jnp.abs((-b + disc) / (2.0 * a))

    beta = jnp.where(
        counts_row < n_lb,
        jnp.power(_BETA, 1.0 / (n_lb - counts_row)),
        jnp.where(counts_row > n_ub,
                  jnp.power(_BETA, 1.0 / (counts_row - n_ub)),
                  _BETA))
    eff = 1.0 - jnp.power(beta, counts_row)
    w_cls = (1.0 - beta) / eff
    w_cls = w_cls / jnp.sum(w_cls) * k
    w_n = jnp.dot(w_cls, oht,
                  preferred_element_type=jnp.float32)

    total = jax.lax.dot_general(w_n, bcesum, (((1,), (0,)), ((), ())),
                                preferred_element_type=jnp.float32)
    out_ref[...] = total[0, 0] / (b_sz * k)


def kernel(logits, labels, features, sample_num_per_cls):
    out = pl.pallas_call(
        _homovar_kernel,
        out_shape=jax.ShapeDtypeStruct((), jnp.float32),
        out_specs=pl.BlockSpec(memory_space=pltpu.SMEM),
    )(logits, labels.astype(jnp.int32), sample_num_per_cls, features)
    return out


# speedup vs baseline: 1.1223x; 1.0357x over previous
import jax
import jax.numpy as jnp
from jax.experimental import pallas as pl

_F_SCORE = 1.2447
_BETA = 0.999


def _homovar_kernel(logits_ref, labels_ref, features_ref, counts_ref, out_ref):
    f = features_ref[:]
    lab_row = labels_ref[:].reshape(1, -1)
    counts_col = counts_ref[:].reshape(-1, 1)
    b_sz = f.shape[0]
    k = counts_col.shape[0]

    lab_col = lab_row.T
    oh = (lab_col == jax.lax.broadcasted_iota(jnp.int32, (b_sz, k), 1)
          ).astype(jnp.float32)
    oht = (lab_row == jax.lax.broadcasted_iota(jnp.int32, (k, b_sz), 0)
           ).astype(jnp.float32)

    inv_counts = 1.0 / counts_col
    segsum = jnp.dot(oht, f, preferred_element_type=jnp.float32)
    g = jnp.dot(oh, segsum, preferred_element_type=jnp.float32)
    invc_n = jnp.dot(oh, inv_counts, preferred_element_type=jnp.float32)
    m = g * invc_n

    z = jnp.sum(jnp.abs(f - m) * (f != 0.0).astype(jnp.float32),
                axis=1, keepdims=True)

    s = jnp.dot(oht, z, preferred_element_type=jnp.float32)
    zi_mean = s * inv_counts
    z_mean = jnp.sum(zi_mean) / k
    n_total = jnp.sum(counts_col)

    zi_g = jnp.dot(oh, zi_mean, preferred_element_type=jnp.float32)
    ssw = jnp.sum((z - zi_g) ** 2 *
                  (z != 0.0).astype(jnp.float32)) / (n_total - k)
    sb = (zi_mean - z_mean) ** 2 * counts_col
    ssb = jnp.sum(sb) / (k - 1)

    cq = _F_SCORE * ssw * (k - 1) - (ssb * (k - 1) - sb)
    a = z_mean ** 2
    b = -(2.0 * z_mean * s + cq)
    cc = s ** 2
    disc = jnp.sqrt(b * b - 4.0 * a * cc)
    n_lb = jnp.abs((-b - disc) / (2.0 * a))
    n_ub = jnp.abs((-b + disc) / (2.0 * a))

    beta = jnp.where(
        counts_col < n_lb,
        jnp.power(_BETA, 1.0 / (n_lb - counts_col)),
        jnp.where(counts_col > n_ub,
                  jnp.power(_BETA, 1.0 / (counts_col - n_ub)),
                  _BETA))
    eff = 1.0 - jnp.power(beta, counts_col)
    w_cls = (1.0 - beta) / eff
    w_cls = w_cls / jnp.sum(w_cls) * k
    w_n = jnp.dot(oh, w_cls, preferred_element_type=jnp.float32)

    lg = logits_ref[:]
    mx = jnp.max(lg, axis=1, keepdims=True)
    e = jnp.exp(lg - mx)
    pred = e / jnp.sum(e, axis=1, keepdims=True)
    log_p = jnp.maximum(jnp.log(pred), -100.0)
    log_1mp = jnp.maximum(jnp.log(1.0 - pred), -100.0)
    bce = -(oh * log_p + (1.0 - oh) * log_1mp)
    total = jnp.sum(w_n * bce, axis=None, keepdims=True)
    out_ref[:, :] = total / (b_sz * k)


def kernel(logits, labels, features, sample_num_per_cls):
    out = pl.pallas_call(
        _homovar_kernel,
        out_shape=jax.ShapeDtypeStruct((1, 1), jnp.float32),
    )(logits, labels.astype(jnp.int32), features, sample_num_per_cls)
    return out[0, 0]
